# Initial kernel scaffold; baseline (speedup 1.0000x reference)
import jax
import jax.numpy as jnp
from jax.experimental import pallas as pl


def kernel(inputs_3, outputs_0, outputs_1, outputs_2, outputs_3, pts, epoch):
    raise NotImplementedError("write your pallas kernel here")


# speedup vs baseline: 4.3267x; 4.3267x over previous
import jax
import jax.numpy as jnp
from jax.experimental import pallas as pl

KNN = 6
_BIG = 3.0e38


def _sqn(x):
    return jnp.sum(x * x, axis=-1)


def _xdist(x, y, yy):
    xy = jax.lax.dot_general(x, y, (((1,), (1,)), ((), ())),
                             preferred_element_type=jnp.float32)
    return _sqn(x)[:, None] + yy[None, :] - 2.0 * xy


def _bnorm(x):
    n = jnp.sqrt(_sqn(x) + 1e-08)[:, None]
    return x / (n + 1e-10)


def _cos(a, b):
    num = jnp.sum(a * b, axis=-1)
    na = jnp.sqrt(_sqn(a))
    nb = jnp.sqrt(_sqn(b))
    return num / jnp.maximum(na * nb, 1e-08)


def _chamfer_body(gt_ref, up_ref, off_ref, upr_ref, offr_ref, upc_ref, offc_ref):
    u = pl.program_id(1)
    gt = gt_ref[0]
    yy = _sqn(gt)
    for x_ref, r_ref, c_ref in ((up_ref, upr_ref, upc_ref),
                                (off_ref, offr_ref, offc_ref)):
        d = _xdist(x_ref[0], gt, yy)
        r_ref[0, 0] = jnp.min(d, axis=1)
        cm = jnp.min(d, axis=0)

        @pl.when(u == 0)
        def _():
            c_ref[0, 0] = cm

        @pl.when(u != 0)
        def _():
            c_ref[0, 0] = jnp.minimum(c_ref[0, 0], cm)


def _nearest_body(pts_ref, gt_ref, nor_ref, out_ref):
    gt = gt_ref[0]
    d = _xdist(pts_ref[0], gt, _sqn(gt))
    m = jnp.min(d, axis=1)
    jidx = jax.lax.broadcasted_iota(jnp.int32, d.shape, 1)
    idx = jnp.min(jnp.where(d == m[:, None], jidx, d.shape[1]), axis=1)
    onehot = (idx[:, None] == jidx).astype(jnp.float32)
    out_ref[0] = jax.lax.dot_general(onehot, nor_ref[0], (((1,), (0,)), ((), ())),
                                     preferred_element_type=jnp.float32)


def _smooth_body(pts_ref, norgt_ref, o0_ref, o1_ref,
                 sm_ref, lnor_ref, lno_ref):
    p = pts_ref[0]
    nor_gt = norgt_ref[0]
    ori_pre = _bnorm(o0_ref[0])
    nor_pre = _bnorm(o1_ref[0])
    d_n = jnp.sum(nor_gt * ori_pre, axis=1)[:, None]
    ori_pro = _bnorm(ori_pre - nor_gt * d_n)
    lnor_ref[0, 0] = 1.0 - jnp.abs(_cos(nor_gt, nor_pre))
    lno_ref[0, 0] = jnp.abs(d_n[:, 0])
    ox, oy, oz = ori_pro[:, 0], ori_pro[:, 1], ori_pro[:, 2]
    nx, ny, nz = nor_gt[:, 0], nor_gt[:, 1], nor_gt[:, 2]
    ori_rot = jnp.stack([oy * nz - oz * ny,
                         oz * nx - ox * nz,
                         ox * ny - oy * nx], axis=1)
    d = _xdist(p, p, _sqn(p))
    n = d.shape[0]
    jidx = jax.lax.broadcasted_iota(jnp.int32, d.shape, 1)
    acc = jnp.zeros((n,), jnp.float32)
    for _ in range(KNN):
        m = jnp.min(d, axis=1)
        a = jnp.min(jnp.where(d == m[:, None], jidx, n), axis=1)
        hit = a[:, None] == jidx
        d = jnp.where(hit, _BIG, d)
        oh = hit.astype(jnp.float32)
        g_ori = jax.lax.dot_general(oh, ori_pro, (((1,), (0,)), ((), ())),
                                    preferred_element_type=jnp.float32)
        g_nor = jax.lax.dot_general(oh, nor_gt, (((1,), (0,)), ((), ())),
                                    preferred_element_type=jnp.float32)
        ndiff = jnp.sum(g_nor * nor_gt, axis=1)
        w = jnp.where(jnp.exp(-ndiff / 0.3) * 10.0 + 1.0 < 4.0, 1.0, 5.0)
        c0 = 1.0 - jnp.abs(_cos(g_ori, ori_pro))
        c1 = 1.0 - jnp.abs(_cos(g_ori, ori_rot))
        acc = acc + w * jnp.minimum(c0, c1)
    sm_ref[0, 0] = acc


def kernel(inputs_3, outputs_0, outputs_1, outputs_2, outputs_3, pts, epoch):
    B, N, _ = pts.shape
    M = inputs_3.shape[1]
    U = outputs_2.shape[1]
    f32 = jnp.float32
    xyz_gt = inputs_3[..., 0:3]
    xyz_nor = inputs_3[..., 3:6]
    pts3 = pts[..., 0:3]

    TU = 256
    nu = U // TU
    upr, offr, upc, offc = pl.pallas_call(
        _chamfer_body,
        grid=(B, nu),
        in_specs=[
            pl.BlockSpec((1, M, 3), lambda b, u: (b, 0, 0)),
            pl.BlockSpec((1, TU, 3), lambda b, u: (b, u, 0)),
            pl.BlockSpec((1, TU, 3), lambda b, u: (b, u, 0)),
        ],
        out_specs=[
            pl.BlockSpec((1, 1, TU), lambda b, u: (b * nu + u, 0, 0)),
            pl.BlockSpec((1, 1, TU), lambda b, u: (b * nu + u, 0, 0)),
            pl.BlockSpec((1, 1, M), lambda b, u: (b, 0, 0)),
            pl.BlockSpec((1, 1, M), lambda b, u: (b, 0, 0)),
        ],
        out_shape=[
            jax.ShapeDtypeStruct((B * nu, 1, TU), f32),
            jax.ShapeDtypeStruct((B * nu, 1, TU), f32),
            jax.ShapeDtypeStruct((B, 1, M), f32),
            jax.ShapeDtypeStruct((B, 1, M), f32),
        ],
    )(xyz_gt, outputs_2, outputs_3)

    TN = 256
    nn = N // TN
    nor_gt = pl.pallas_call(
        _nearest_body,
        grid=(B, nn),
        in_specs=[
            pl.BlockSpec((1, TN, 3), lambda b, i: (b, i, 0)),
            pl.BlockSpec((1, M, 3), lambda b, i: (b, 0, 0)),
            pl.BlockSpec((1, M, 3), lambda b, i: (b, 0, 0)),
        ],
        out_specs=pl.BlockSpec((1, TN, 3), lambda b, i: (b, i, 0)),
        out_shape=jax.ShapeDtypeStruct((B, N, 3), f32),
    )(pts3, xyz_gt, xyz_nor)

    sm, lnor, lno = pl.pallas_call(
        _smooth_body,
        grid=(B,),
        in_specs=[
            pl.BlockSpec((1, N, 3), lambda b: (b, 0, 0)),
            pl.BlockSpec((1, N, 3), lambda b: (b, 0, 0)),
            pl.BlockSpec((1, N, 3), lambda b: (b, 0, 0)),
            pl.BlockSpec((1, N, 3), lambda b: (b, 0, 0)),
        ],
        out_specs=[
            pl.BlockSpec((1, 1, N), lambda b: (b, 0, 0)),
            pl.BlockSpec((1, 1, N), lambda b: (b, 0, 0)),
            pl.BlockSpec((1, 1, N), lambda b: (b, 0, 0)),
        ],
        out_shape=[
            jax.ShapeDtypeStruct((B, 1, N), f32),
            jax.ShapeDtypeStruct((B, 1, N), f32),
            jax.ShapeDtypeStruct((B, 1, N), f32),
        ],
    )(pts3, nor_gt, outputs_0, outputs_1)

    loss_smooth = jnp.mean(sm) / KNN
    loss_nor = jnp.mean(lnor)
    loss_nor_ori = jnp.mean(lno)
    loss_charm_offset = jnp.mean(offr) + jnp.mean(offc)
    loss_charm = jnp.mean(upr) + jnp.mean(upc)
    loss_cd = loss_charm + 0.4 * loss_charm_offset
    loss = loss_smooth + loss_nor + 0.1 * loss_nor_ori + 200.0 * loss_cd
    return (loss, loss_smooth, loss_nor, loss_charm_offset, loss_nor_ori,
            loss_charm, loss_charm)


# speedup vs baseline: 4.5244x; 1.0457x over previous
import jax
import jax.numpy as jnp
from jax.experimental import pallas as pl

KNN = 6
_BIG = 3.0e38


def _sqn(x):
    return jnp.sum(x * x, axis=-1)


def _dot(x, y):
    return jax.lax.dot_general(x, y, (((1,), (1,)), ((), ())),
                               preferred_element_type=jnp.float32)


def _mm(a, b):
    return jax.lax.dot_general(a, b, (((1,), (0,)), ((), ())),
                               preferred_element_type=jnp.float32)


def _bnorm(x):
    n = jnp.sqrt(_sqn(x) + 1e-08)[:, None]
    return x / (n + 1e-10)


def _cos(a, b):
    num = jnp.sum(a * b, axis=-1)
    na = jnp.sqrt(_sqn(a))
    nb = jnp.sqrt(_sqn(b))
    return num / jnp.maximum(na * nb, 1e-08)


def _chamfer_body(gt_ref, up_ref, off_ref, upr_ref, offr_ref, upc_ref, offc_ref):
    u = pl.program_id(1)
    gt = gt_ref[0]
    yy = _sqn(gt)
    for x_ref, r_ref, c_ref in ((up_ref, upr_ref, upc_ref),
                                (off_ref, offr_ref, offc_ref)):
        x = x_ref[0]
        d = _sqn(x)[:, None] + yy[None, :] - 2.0 * _dot(x, gt)
        r_ref[0, 0] = jnp.min(d, axis=1)
        cm = jnp.min(d, axis=0)

        @pl.when(u == 0)
        def _():
            c_ref[0, 0] = cm

        @pl.when(u != 0)
        def _():
            c_ref[0, 0] = jnp.minimum(c_ref[0, 0], cm)


def _nearest_body(pts_ref, gt_ref, nor_ref, out_ref):
    gt = gt_ref[0]
    x = pts_ref[0]
    s = _sqn(x)[:, None] + _sqn(gt)[None, :] - 2.0 * _dot(x, gt)
    m = jnp.min(s, axis=1)
    jidx = jax.lax.broadcasted_iota(jnp.int32, s.shape, 1)
    idx = jnp.min(jnp.where(s == m[:, None], jidx, s.shape[1]), axis=1)
    onehot = (idx[:, None] == jidx).astype(jnp.float32)
    out_ref[0] = _mm(onehot, nor_ref[0])


def _smooth_body(pts_ref, norgt_ref, o0_ref, o1_ref,
                 sm_ref, lnor_ref, lno_ref):
    p = pts_ref[0]
    nor_gt = norgt_ref[0]
    ori_pre = _bnorm(o0_ref[0])
    nor_pre = _bnorm(o1_ref[0])
    d_n = jnp.sum(nor_gt * ori_pre, axis=1)[:, None]
    ori_pro = _bnorm(ori_pre - nor_gt * d_n)
    lnor_ref[0, 0] = 1.0 - jnp.abs(_cos(nor_gt, nor_pre))
    lno_ref[0, 0] = jnp.abs(d_n[:, 0])
    ox, oy, oz = ori_pro[:, 0], ori_pro[:, 1], ori_pro[:, 2]
    nx, ny, nz = nor_gt[:, 0], nor_gt[:, 1], nor_gt[:, 2]
    ori_rot = jnp.stack([oy * nz - oz * ny,
                         oz * nx - ox * nz,
                         ox * ny - oy * nx], axis=1)
    pp = _sqn(p)
    s = pp[:, None] + pp[None, :] - 2.0 * _dot(p, p)
    jidx = jax.lax.broadcasted_iota(jnp.int32, s.shape, 1)
    rhs = jnp.concatenate([ori_pro, nor_gt], axis=1)
    n = s.shape[0]
    acc = jnp.zeros((n,), jnp.float32)
    for _ in range(KNN):
        m = jnp.min(s, axis=1)
        a = jnp.min(jnp.where(s == m[:, None], jidx, n), axis=1)
        hit = a[:, None] == jidx
        s = jnp.where(hit, _BIG, s)
        g = _mm(hit.astype(jnp.float32), rhs)
        g_ori = g[:, 0:3]
        g_nor = g[:, 3:6]
        ndiff = jnp.sum(g_nor * nor_gt, axis=1)
        w = jnp.where(jnp.exp(-ndiff / 0.3) * 10.0 + 1.0 < 4.0, 1.0, 5.0)
        c0 = 1.0 - jnp.abs(_cos(g_ori, ori_pro))
        c1 = 1.0 - jnp.abs(_cos(g_ori, ori_rot))
        acc = acc + w * jnp.minimum(c0, c1)
    sm_ref[0, 0] = acc


def kernel(inputs_3, outputs_0, outputs_1, outputs_2, outputs_3, pts, epoch):
    B, N, _ = pts.shape
    M = inputs_3.shape[1]
    U = outputs_2.shape[1]
    f32 = jnp.float32
    xyz_gt = inputs_3[..., 0:3]
    xyz_nor = inputs_3[..., 3:6]
    pts3 = pts[..., 0:3]

    TU = 512
    nu = U // TU
    upr, offr, upc, offc = pl.pallas_call(
        _chamfer_body,
        grid=(B, nu),
        in_specs=[
            pl.BlockSpec((1, M, 3), lambda b, u: (b, 0, 0)),
            pl.BlockSpec((1, TU, 3), lambda b, u: (b, u, 0)),
            pl.BlockSpec((1, TU, 3), lambda b, u: (b, u, 0)),
        ],
        out_specs=[
            pl.BlockSpec((1, 1, TU), lambda b, u: (b * nu + u, 0, 0)),
            pl.BlockSpec((1, 1, TU), lambda b, u: (b * nu + u, 0, 0)),
            pl.BlockSpec((1, 1, M), lambda b, u: (b, 0, 0)),
            pl.BlockSpec((1, 1, M), lambda b, u: (b, 0, 0)),
        ],
        out_shape=[
            jax.ShapeDtypeStruct((B * nu, 1, TU), f32),
            jax.ShapeDtypeStruct((B * nu, 1, TU), f32),
            jax.ShapeDtypeStruct((B, 1, M), f32),
            jax.ShapeDtypeStruct((B, 1, M), f32),
        ],
    )(xyz_gt, outputs_2, outputs_3)

    TN = 256
    nn = N // TN
    nor_gt = pl.pallas_call(
        _nearest_body,
        grid=(B, nn),
        in_specs=[
            pl.BlockSpec((1, TN, 3), lambda b, i: (b, i, 0)),
            pl.BlockSpec((1, M, 3), lambda b, i: (b, 0, 0)),
            pl.BlockSpec((1, M, 3), lambda b, i: (b, 0, 0)),
        ],
        out_specs=pl.BlockSpec((1, TN, 3), lambda b, i: (b, i, 0)),
        out_shape=jax.ShapeDtypeStruct((B, N, 3), f32),
    )(pts3, xyz_gt, xyz_nor)

    sm, lnor, lno = pl.pallas_call(
        _smooth_body,
        grid=(B,),
        in_specs=[
            pl.BlockSpec((1, N, 3), lambda b: (b, 0, 0)),
            pl.BlockSpec((1, N, 3), lambda b: (b, 0, 0)),
            pl.BlockSpec((1, N, 3), lambda b: (b, 0, 0)),
            pl.BlockSpec((1, N, 3), lambda b: (b, 0, 0)),
        ],
        out_specs=[
            pl.BlockSpec((1, 1, N), lambda b: (b, 0, 0)),
            pl.BlockSpec((1, 1, N), lambda b: (b, 0, 0)),
            pl.BlockSpec((1, 1, N), lambda b: (b, 0, 0)),
        ],
        out_shape=[
            jax.ShapeDtypeStruct((B, 1, N), f32),
            jax.ShapeDtypeStruct((B, 1, N), f32),
            jax.ShapeDtypeStruct((B, 1, N), f32),
        ],
    )(pts3, nor_gt, outputs_0, outputs_1)

    loss_smooth = jnp.mean(sm) / KNN
    loss_nor = jnp.mean(lnor)
    loss_nor_ori = jnp.mean(lno)
    loss_charm_offset = jnp.mean(offr) + jnp.mean(offc)
    loss_charm = jnp.mean(upr) + jnp.mean(upc)
    loss_cd = loss_charm + 0.4 * loss_charm_offset
    loss = loss_smooth + loss_nor + 0.1 * loss_nor_ori + 200.0 * loss_cd
    return (loss, loss_smooth, loss_nor, loss_charm_offset, loss_nor_ori,
            loss_charm, loss_charm)


# speedup vs baseline: 4.9018x; 1.0834x over previous
import jax
import jax.numpy as jnp
from jax.experimental import pallas as pl
from jax.experimental.pallas import tpu as pltpu

KNN = 6
_BIG = 3.0e38


def _sqn(x):
    return jnp.sum(x * x, axis=-1)


def _dot(x, y):
    return jax.lax.dot_general(x, y, (((1,), (1,)), ((), ())),
                               preferred_element_type=jnp.float32)


def _mm(a, b):
    return jax.lax.dot_general(a, b, (((1,), (0,)), ((), ())),
                               preferred_element_type=jnp.float32)


def _bnorm(x):
    n = jnp.sqrt(_sqn(x) + 1e-08)[:, None]
    return x / (n + 1e-10)


def _cos(a, b):
    num = jnp.sum(a * b, axis=-1)
    na = jnp.sqrt(_sqn(a))
    nb = jnp.sqrt(_sqn(b))
    return num / jnp.maximum(na * nb, 1e-08)


def _chamfer_body(in3_ref, up_ref, off_ref, upr_ref, offr_ref, upc_ref, offc_ref):
    u = pl.program_id(1)
    gt = in3_ref[0][:, 0:3]
    yy = _sqn(gt)
    for x_ref, r_ref, c_ref in ((up_ref, upr_ref, upc_ref),
                                (off_ref, offr_ref, offc_ref)):
        x = x_ref[0]
        d = _sqn(x)[:, None] + yy[None, :] - 2.0 * _dot(x, gt)
        r_ref[0, 0] = jnp.min(d, axis=1)
        cm = jnp.min(d, axis=0)

        @pl.when(u == 0)
        def _():
            c_ref[0, 0] = cm

        @pl.when(u != 0)
        def _():
            c_ref[0, 0] = jnp.minimum(c_ref[0, 0], cm)


def _nearest_body(pts_ref, in3_ref, out_ref):
    in3 = in3_ref[0]
    gt = in3[:, 0:3]
    x = pts_ref[0]
    s = _sqn(x)[:, None] + _sqn(gt)[None, :] - 2.0 * _dot(x, gt)
    m = jnp.min(s, axis=1)
    jidx = jax.lax.broadcasted_iota(jnp.int32, s.shape, 1)
    idx = jnp.min(jnp.where(s == m[:, None], jidx, s.shape[1]), axis=1)
    onehot = (idx[:, None] == jidx).astype(jnp.float32)
    out_ref[0] = _mm(onehot, in3[:, 3:6])


def _smooth_body(pts_ref, norgt_ref, o0_ref, o1_ref,
                 sm_ref, lnor_ref, lno_ref):
    p = pts_ref[0]
    nor_gt = norgt_ref[0]
    ori_pre = _bnorm(o0_ref[0])
    nor_pre = _bnorm(o1_ref[0])
    d_n = jnp.sum(nor_gt * ori_pre, axis=1)[:, None]
    ori_pro = _bnorm(ori_pre - nor_gt * d_n)
    lnor_ref[0, 0] = 1.0 - jnp.abs(_cos(nor_gt, nor_pre))
    lno_ref[0, 0] = jnp.abs(d_n[:, 0])
    ox, oy, oz = ori_pro[:, 0], ori_pro[:, 1], ori_pro[:, 2]
    nx, ny, nz = nor_gt[:, 0], nor_gt[:, 1], nor_gt[:, 2]
    ori_rot = jnp.stack([oy * nz - oz * ny,
                         oz * nx - ox * nz,
                         ox * ny - oy * nx], axis=1)
    pp = _sqn(p)
    s = pp[:, None] + pp[None, :] - 2.0 * _dot(p, p)
    jidx = jax.lax.broadcasted_iota(jnp.int32, s.shape, 1)
    rhs = jnp.concatenate([ori_pro, nor_gt], axis=1)
    n = s.shape[0]
    acc = jnp.zeros((n,), jnp.float32)
    for _ in range(KNN):
        m = jnp.min(s, axis=1)
        a = jnp.min(jnp.where(s == m[:, None], jidx, n), axis=1)
        hit = a[:, None] == jidx
        s = jnp.where(hit, _BIG, s)
        g = _mm(hit.astype(jnp.float32), rhs)
        g_ori = g[:, 0:3]
        g_nor = g[:, 3:6]
        ndiff = jnp.sum(g_nor * nor_gt, axis=1)
        w = jnp.where(jnp.exp(-ndiff / 0.3) * 10.0 + 1.0 < 4.0, 1.0, 5.0)
        c0 = 1.0 - jnp.abs(_cos(g_ori, ori_pro))
        c1 = 1.0 - jnp.abs(_cos(g_ori, ori_rot))
        acc = acc + w * jnp.minimum(c0, c1)
    sm_ref[0, 0] = acc


def _finish_body(upr_ref, offr_ref, upc_ref, offc_ref, sm_ref, lnor_ref,
                 lno_ref, loss_ref, lsm_ref, lnr_ref, lco_ref, lno_out_ref,
                 lch_ref):
    nu = upr_ref.shape[0] * upr_ref.shape[2]
    nm = upc_ref.shape[0] * upc_ref.shape[2]
    nn = sm_ref.shape[0] * sm_ref.shape[2]
    loss_smooth = jnp.sum(sm_ref[...]) / (nn * KNN)
    loss_nor = jnp.sum(lnor_ref[...]) / nn
    loss_nor_ori = jnp.sum(lno_ref[...]) / nn
    loss_charm_offset = jnp.sum(offr_ref[...]) / nu + jnp.sum(offc_ref[...]) / nm
    loss_charm = jnp.sum(upr_ref[...]) / nu + jnp.sum(upc_ref[...]) / nm
    loss_cd = loss_charm + 0.4 * loss_charm_offset
    loss = loss_smooth + loss_nor + 0.1 * loss_nor_ori + 200.0 * loss_cd
    loss_ref[0, 0] = loss
    lsm_ref[0, 0] = loss_smooth
    lnr_ref[0, 0] = loss_nor
    lco_ref[0, 0] = loss_charm_offset
    lno_out_ref[0, 0] = loss_nor_ori
    lch_ref[0, 0] = loss_charm


def kernel(inputs_3, outputs_0, outputs_1, outputs_2, outputs_3, pts, epoch):
    B, N, _ = pts.shape
    M = inputs_3.shape[1]
    U = outputs_2.shape[1]
    f32 = jnp.float32

    TU = 512
    nu = U // TU
    upr, offr, upc, offc = pl.pallas_call(
        _chamfer_body,
        grid=(B, nu),
        in_specs=[
            pl.BlockSpec((1, M, 6), lambda b, u: (b, 0, 0)),
            pl.BlockSpec((1, TU, 3), lambda b, u: (b, u, 0)),
            pl.BlockSpec((1, TU, 3), lambda b, u: (b, u, 0)),
        ],
        out_specs=[
            pl.BlockSpec((1, 1, TU), lambda b, u: (b * nu + u, 0, 0)),
            pl.BlockSpec((1, 1, TU), lambda b, u: (b * nu + u, 0, 0)),
            pl.BlockSpec((1, 1, M), lambda b, u: (b, 0, 0)),
            pl.BlockSpec((1, 1, M), lambda b, u: (b, 0, 0)),
        ],
        out_shape=[
            jax.ShapeDtypeStruct((B * nu, 1, TU), f32),
            jax.ShapeDtypeStruct((B * nu, 1, TU), f32),
            jax.ShapeDtypeStruct((B, 1, M), f32),
            jax.ShapeDtypeStruct((B, 1, M), f32),
        ],
    )(inputs_3, outputs_2, outputs_3)

    TN = 256
    nn = N // TN
    nor_gt = pl.pallas_call(
        _nearest_body,
        grid=(B, nn),
        in_specs=[
            pl.BlockSpec((1, TN, 3), lambda b, i: (b, i, 0)),
            pl.BlockSpec((1, M, 6), lambda b, i: (b, 0, 0)),
        ],
        out_specs=pl.BlockSpec((1, TN, 3), lambda b, i: (b, i, 0)),
        out_shape=jax.ShapeDtypeStruct((B, N, 3), f32),
    )(pts, inputs_3)

    sm, lnor, lno = pl.pallas_call(
        _smooth_body,
        grid=(B,),
        in_specs=[
            pl.BlockSpec((1, N, 3), lambda b: (b, 0, 0)),
            pl.BlockSpec((1, N, 3), lambda b: (b, 0, 0)),
            pl.BlockSpec((1, N, 3), lambda b: (b, 0, 0)),
            pl.BlockSpec((1, N, 3), lambda b: (b, 0, 0)),
        ],
        out_specs=[
            pl.BlockSpec((1, 1, N), lambda b: (b, 0, 0)),
            pl.BlockSpec((1, 1, N), lambda b: (b, 0, 0)),
            pl.BlockSpec((1, 1, N), lambda b: (b, 0, 0)),
        ],
        out_shape=[
            jax.ShapeDtypeStruct((B, 1, N), f32),
            jax.ShapeDtypeStruct((B, 1, N), f32),
            jax.ShapeDtypeStruct((B, 1, N), f32),
        ],
    )(pts, nor_gt, outputs_0, outputs_1)

    scalar_out = pl.BlockSpec(memory_space=pltpu.SMEM)
    outs = pl.pallas_call(
        _finish_body,
        in_specs=[pl.BlockSpec(a.shape, lambda: (0,) * 3)
                  for a in (upr, offr, upc, offc, sm, lnor, lno)],
        out_specs=[scalar_out] * 6,
        out_shape=[jax.ShapeDtypeStruct((1, 1), f32)] * 6,
    )(upr, offr, upc, offc, sm, lnor, lno)
    (loss, loss_smooth, loss_nor, loss_charm_offset,
     loss_nor_ori, loss_charm) = [o[0, 0] for o in outs]
    return (loss, loss_smooth, loss_nor, loss_charm_offset, loss_nor_ori,
            loss_charm, loss_charm)
